# baseline (speedup 1.0000x reference)
import jax
import jax.numpy as jnp
from jax import lax
from jax.experimental import pallas as pl
from jax.experimental.pallas import tpu as pltpu
from jax.experimental.pallas import tpu_sc as plsc

BATCH = 16384
DIM = 2048
LANES = 16

NUM_CORES = 2
NUM_SUBCORES = 16
NW = NUM_CORES * NUM_SUBCORES

ROWS_PER_W = BATCH // NW
CHUNK_R = 8
CHUNKS = ROWS_PER_W // CHUNK_R
JCHUNKS = DIM // LANES


def _body(x_hbm, perm_hbm, out_hbm, perm_v, in_v, out_v):
    wid = lax.axis_index("s") * NUM_CORES + lax.axis_index("c")
    base = wid * ROWS_PER_W * DIM

    pltpu.sync_copy(perm_hbm, perm_v)

    def chunk_body(ch, carry):
        off = base + ch * (CHUNK_R * DIM)
        pltpu.sync_copy(x_hbm.at[pl.ds(off, CHUNK_R * DIM)], in_v)

        def j_body(j, carry2):
            pc = perm_v[pl.ds(j * LANES, LANES)]
            for r in range(CHUNK_R):
                idx = pc + jnp.int32(r * DIM)
                v = plsc.load_gather(in_v, [idx])
                out_v[pl.ds(r * DIM + j * LANES, LANES)] = v
            return carry2

        lax.fori_loop(0, JCHUNKS, j_body, 0)
        pltpu.sync_copy(out_v, out_hbm.at[pl.ds(off, CHUNK_R * DIM)])
        return carry

    lax.fori_loop(0, CHUNKS, chunk_body, 0)


@jax.jit
def _permute(x_flat, perm):
    mesh = plsc.VectorSubcoreMesh(core_axis_name="c", subcore_axis_name="s")
    f = pl.kernel(
        _body,
        out_type=jax.ShapeDtypeStruct((BATCH * DIM,), jnp.float32),
        mesh=mesh,
        scratch_types=[
            pltpu.VMEM((DIM,), jnp.int32),
            pltpu.VMEM((CHUNK_R * DIM,), jnp.float32),
            pltpu.VMEM((CHUNK_R * DIM,), jnp.float32),
        ],
        compiler_params=pltpu.CompilerParams(
            use_tc_tiling_on_sc=False, needs_layout_passes=False
        ),
    )
    return f(x_flat, perm)


def kernel(x, perm):
    out_flat = _permute(x.reshape(-1), perm.astype(jnp.int32))
    out = out_flat.reshape(BATCH, DIM)
    logdet = jnp.zeros(x.shape[0], dtype=x.dtype)
    return (out, logdet)


# speedup vs baseline: 1.9147x; 1.9147x over previous
import jax
import jax.numpy as jnp
from jax import lax
from jax.experimental import pallas as pl
from jax.experimental.pallas import tpu as pltpu
from jax.experimental.pallas import tpu_sc as plsc

BATCH = 16384
DIM = 2048
LANES = 16

NUM_CORES = 2
NUM_SUBCORES = 16
NW = NUM_CORES * NUM_SUBCORES

ROWS_PER_W = BATCH // NW
CHUNK_R = 8
CSZ = CHUNK_R * DIM
CHUNKS = ROWS_PER_W // CHUNK_R
NPAIR = CHUNKS // 2
JCHUNKS = DIM // LANES


def _body(x_hbm, perm_hbm, out_hbm, perm_v, in0, in1, out0, out1,
          is0, is1, os0, os1):
    wid = lax.axis_index("s") * NUM_CORES + lax.axis_index("c")
    base = wid * ROWS_PER_W * DIM

    ins, outs = [in0, in1], [out0, out1]
    isems, osems = [is0, is1], [os0, os1]

    def in_copy(b, ch):
        return pltpu.make_async_copy(
            x_hbm.at[pl.ds(base + ch * CSZ, CSZ)], ins[b], isems[b])

    def out_copy(b, ch):
        return pltpu.make_async_copy(
            outs[b], out_hbm.at[pl.ds(base + ch * CSZ, CSZ)], osems[b])

    def gather(in_b, out_b):
        @plsc.parallel_loop(0, JCHUNKS, unroll=4)
        def _(jc):
            pc = perm_v[pl.ds(jc * LANES, LANES)]
            for r in range(CHUNK_R):
                v = plsc.load_gather(in_b, [pc + jnp.int32(r * DIM)])
                out_b[pl.ds(r * DIM + jc * LANES, LANES)] = v

    pltpu.sync_copy(perm_hbm, perm_v)

    in_copy(0, 0).start()
    in_copy(1, 1).start()

    def pair_body(g, carry):
        for b in range(2):
            ch = 2 * g + b
            in_copy(b, ch).wait()

            @pl.when(g > 0)
            def _():
                out_copy(b, ch - 2).wait()

            gather(ins[b], outs[b])
            out_copy(b, ch).start()

            @pl.when(g < NPAIR - 1)
            def _():
                in_copy(b, ch + 2).start()

        return carry

    lax.fori_loop(0, NPAIR, pair_body, 0)

    out_copy(0, CHUNKS - 2).wait()
    out_copy(1, CHUNKS - 1).wait()


@jax.jit
def _permute(x_flat, perm):
    mesh = plsc.VectorSubcoreMesh(core_axis_name="c", subcore_axis_name="s")
    f = pl.kernel(
        _body,
        out_type=jax.ShapeDtypeStruct((BATCH * DIM,), jnp.float32),
        mesh=mesh,
        scratch_types=[
            pltpu.VMEM((DIM,), jnp.int32),
            pltpu.VMEM((CSZ,), jnp.float32),
            pltpu.VMEM((CSZ,), jnp.float32),
            pltpu.VMEM((CSZ,), jnp.float32),
            pltpu.VMEM((CSZ,), jnp.float32),
            pltpu.SemaphoreType.DMA,
            pltpu.SemaphoreType.DMA,
            pltpu.SemaphoreType.DMA,
            pltpu.SemaphoreType.DMA,
        ],
        compiler_params=pltpu.CompilerParams(
            use_tc_tiling_on_sc=False, needs_layout_passes=False
        ),
    )
    return f(x_flat, perm)


def kernel(x, perm):
    out_flat = _permute(x.reshape(-1), perm.astype(jnp.int32))
    out = out_flat.reshape(BATCH, DIM)
    logdet = jnp.zeros(x.shape[0], dtype=x.dtype)
    return (out, logdet)


# speedup vs baseline: 5.5953x; 2.9222x over previous
import jax
import jax.numpy as jnp
from jax import lax
from jax.experimental import pallas as pl
from jax.experimental.pallas import tpu as pltpu
from jax.experimental.pallas import tpu_sc as plsc

BATCH = 16384
DIM = 2048
LANES = 16

NUM_CORES = 2
NUM_SUBCORES = 16
NW = NUM_CORES * NUM_SUBCORES

ROWS_PER_W = BATCH // NW
CHUNK_R = 8
CHUNKS = ROWS_PER_W // CHUNK_R
NPAIR = CHUNKS // 2
JCHUNKS = DIM // LANES


def _body(x_hbm, perm_hbm, out_hbm, perm_v, in0, in1, out0, out1,
          is0, is1, os0, os1):
    wid = lax.axis_index("s") * NUM_CORES + lax.axis_index("c")
    base = wid * ROWS_PER_W

    ins, outs = [in0, in1], [out0, out1]
    isems, osems = [is0, is1], [os0, os1]

    def in_copy(b, ch):
        return pltpu.make_async_copy(
            x_hbm.at[pl.ds(base + ch * CHUNK_R, CHUNK_R)], ins[b], isems[b])

    def out_copy(b, ch):
        return pltpu.make_async_copy(
            outs[b], out_hbm.at[pl.ds(base + ch * CHUNK_R, CHUNK_R)],
            osems[b])

    def gather(in_b, out_b):
        @plsc.parallel_loop(0, JCHUNKS, unroll=4)
        def _(jc):
            pc = perm_v[pl.ds(jc * LANES, LANES)]
            for r in range(CHUNK_R):
                ridx = jnp.full((LANES,), r, dtype=jnp.int32)
                v = plsc.load_gather(in_b, [ridx, pc])
                out_b[r, pl.ds(jc * LANES, LANES)] = v

    pltpu.sync_copy(perm_hbm, perm_v)

    in_copy(0, 0).start()
    in_copy(1, 1).start()

    def pair_body(g, carry):
        for b in range(2):
            ch = 2 * g + b
            in_copy(b, ch).wait()

            @pl.when(g > 0)
            def _():
                out_copy(b, ch - 2).wait()

            gather(ins[b], outs[b])
            out_copy(b, ch).start()

            @pl.when(g < NPAIR - 1)
            def _():
                in_copy(b, ch + 2).start()

        return carry

    lax.fori_loop(0, NPAIR, pair_body, 0)

    out_copy(0, CHUNKS - 2).wait()
    out_copy(1, CHUNKS - 1).wait()


@jax.jit
def _permute(x, perm):
    mesh = plsc.VectorSubcoreMesh(core_axis_name="c", subcore_axis_name="s")
    f = pl.kernel(
        _body,
        out_type=jax.ShapeDtypeStruct((BATCH, DIM), jnp.float32),
        mesh=mesh,
        scratch_types=[
            pltpu.VMEM((DIM,), jnp.int32),
            pltpu.VMEM((CHUNK_R, DIM), jnp.float32),
            pltpu.VMEM((CHUNK_R, DIM), jnp.float32),
            pltpu.VMEM((CHUNK_R, DIM), jnp.float32),
            pltpu.VMEM((CHUNK_R, DIM), jnp.float32),
            pltpu.SemaphoreType.DMA,
            pltpu.SemaphoreType.DMA,
            pltpu.SemaphoreType.DMA,
            pltpu.SemaphoreType.DMA,
        ],
        compiler_params=pltpu.CompilerParams(
            use_tc_tiling_on_sc=True, needs_layout_passes=False
        ),
    )
    return f(x, perm)


def kernel(x, perm):
    out = _permute(x, perm.astype(jnp.int32))
    logdet = jnp.zeros(x.shape[0], dtype=x.dtype)
    return (out, logdet)


# speedup vs baseline: 5.6045x; 1.0016x over previous
import jax
import jax.numpy as jnp
from jax import lax
from jax.experimental import pallas as pl
from jax.experimental.pallas import tpu as pltpu
from jax.experimental.pallas import tpu_sc as plsc

BATCH = 16384
DIM = 2048
LANES = 16

NUM_CORES = 2
NUM_SUBCORES = 16
NW = NUM_CORES * NUM_SUBCORES

ROWS_PER_W = BATCH // NW
CHUNK_R = 8
CHUNKS = ROWS_PER_W // CHUNK_R
NPAIR = CHUNKS // 2
JCHUNKS = DIM // LANES


def _body(x_hbm, perm_hbm, out_hbm, ld_hbm, perm_v, zb, in0, in1, out0, out1,
          is0, is1, os0, os1):
    wid = lax.axis_index("s") * NUM_CORES + lax.axis_index("c")
    base = wid * ROWS_PER_W

    ins, outs = [in0, in1], [out0, out1]
    isems, osems = [is0, is1], [os0, os1]

    def in_copy(b, ch):
        return pltpu.make_async_copy(
            x_hbm.at[pl.ds(base + ch * CHUNK_R, CHUNK_R)], ins[b], isems[b])

    def out_copy(b, ch):
        return pltpu.make_async_copy(
            outs[b], out_hbm.at[pl.ds(base + ch * CHUNK_R, CHUNK_R)],
            osems[b])

    def gather(in_b, out_b):
        @plsc.parallel_loop(0, JCHUNKS, unroll=4)
        def _(jc):
            pc = perm_v[pl.ds(jc * LANES, LANES)]
            for r in range(CHUNK_R):
                ridx = jnp.full((LANES,), r, dtype=jnp.int32)
                v = plsc.load_gather(in_b, [ridx, pc])
                out_b[r, pl.ds(jc * LANES, LANES)] = v

    pltpu.sync_copy(perm_hbm, perm_v)

    in_copy(0, 0).start()
    in_copy(1, 1).start()

    def zero_body(i, carry):
        zb[pl.ds(i * LANES, LANES)] = jnp.zeros((LANES,), jnp.float32)
        return carry

    lax.fori_loop(0, ROWS_PER_W // LANES, zero_body, 0)
    pltpu.sync_copy(zb, ld_hbm.at[pl.ds(base, ROWS_PER_W)])

    def pair_body(g, carry):
        for b in range(2):
            ch = 2 * g + b
            in_copy(b, ch).wait()

            @pl.when(g > 0)
            def _():
                out_copy(b, ch - 2).wait()

            gather(ins[b], outs[b])
            out_copy(b, ch).start()

            @pl.when(g < NPAIR - 1)
            def _():
                in_copy(b, ch + 2).start()

        return carry

    lax.fori_loop(0, NPAIR, pair_body, 0)

    out_copy(0, CHUNKS - 2).wait()
    out_copy(1, CHUNKS - 1).wait()


@jax.jit
def _permute(x, perm):
    mesh = plsc.VectorSubcoreMesh(core_axis_name="c", subcore_axis_name="s")
    f = pl.kernel(
        _body,
        out_type=(
            jax.ShapeDtypeStruct((BATCH, DIM), jnp.float32),
            jax.ShapeDtypeStruct((BATCH,), jnp.float32),
        ),
        mesh=mesh,
        scratch_types=[
            pltpu.VMEM((DIM,), jnp.int32),
            pltpu.VMEM((ROWS_PER_W,), jnp.float32),
            pltpu.VMEM((CHUNK_R, DIM), jnp.float32),
            pltpu.VMEM((CHUNK_R, DIM), jnp.float32),
            pltpu.VMEM((CHUNK_R, DIM), jnp.float32),
            pltpu.VMEM((CHUNK_R, DIM), jnp.float32),
            pltpu.SemaphoreType.DMA,
            pltpu.SemaphoreType.DMA,
            pltpu.SemaphoreType.DMA,
            pltpu.SemaphoreType.DMA,
        ],
        compiler_params=pltpu.CompilerParams(
            use_tc_tiling_on_sc=True, needs_layout_passes=False
        ),
    )
    return f(x, perm)


def kernel(x, perm):
    out, logdet = _permute(x, perm.astype(jnp.int32))
    return (out, logdet)


# speedup vs baseline: 5.7899x; 1.0331x over previous
import jax
import jax.numpy as jnp
from jax import lax
from jax.experimental import pallas as pl
from jax.experimental.pallas import tpu as pltpu
from jax.experimental.pallas import tpu_sc as plsc

BATCH = 16384
DIM = 2048
LANES = 16

NUM_CORES = 2
NUM_SUBCORES = 16
NW = NUM_CORES * NUM_SUBCORES

ROWS_PER_W = BATCH // NW
CHUNK_R = 4
CHUNKS = ROWS_PER_W // CHUNK_R
NBUF = 4
NGROUP = CHUNKS // NBUF
JCHUNKS = DIM // LANES


def _body(x_hbm, perm_hbm, out_hbm, ld_hbm, perm_v, zb, *bufs_and_sems):
    ins = list(bufs_and_sems[0:NBUF])
    outs = list(bufs_and_sems[NBUF:2 * NBUF])
    isems = list(bufs_and_sems[2 * NBUF:3 * NBUF])
    osems = list(bufs_and_sems[3 * NBUF:4 * NBUF])

    wid = lax.axis_index("s") * NUM_CORES + lax.axis_index("c")
    base = wid * ROWS_PER_W

    def in_copy(b, ch):
        return pltpu.make_async_copy(
            x_hbm.at[pl.ds(base + ch * CHUNK_R, CHUNK_R)], ins[b], isems[b])

    def out_copy(b, ch):
        return pltpu.make_async_copy(
            outs[b], out_hbm.at[pl.ds(base + ch * CHUNK_R, CHUNK_R)],
            osems[b])

    def gather(in_b, out_b):
        @plsc.parallel_loop(0, JCHUNKS, unroll=4)
        def _(jc):
            pc = perm_v[pl.ds(jc * LANES, LANES)]
            for r in range(CHUNK_R):
                ridx = jnp.full((LANES,), r, dtype=jnp.int32)
                v = plsc.load_gather(in_b, [ridx, pc])
                out_b[r, pl.ds(jc * LANES, LANES)] = v

    pltpu.sync_copy(perm_hbm, perm_v)

    for b in range(NBUF):
        in_copy(b, b).start()

    def zero_body(i, carry):
        zb[pl.ds(i * LANES, LANES)] = jnp.zeros((LANES,), jnp.float32)
        return carry

    lax.fori_loop(0, ROWS_PER_W // LANES, zero_body, 0)
    pltpu.sync_copy(zb, ld_hbm.at[pl.ds(base, ROWS_PER_W)])

    def group_body(g, carry):
        for b in range(NBUF):
            ch = NBUF * g + b
            in_copy(b, ch).wait()

            @pl.when(g > 0)
            def _():
                out_copy(b, ch - NBUF).wait()

            gather(ins[b], outs[b])
            out_copy(b, ch).start()

            @pl.when(g < NGROUP - 1)
            def _():
                in_copy(b, ch + NBUF).start()

        return carry

    lax.fori_loop(0, NGROUP, group_body, 0)

    for b in range(NBUF):
        out_copy(b, CHUNKS - NBUF + b).wait()


@jax.jit
def _permute(x, perm):
    mesh = plsc.VectorSubcoreMesh(core_axis_name="c", subcore_axis_name="s")
    f = pl.kernel(
        _body,
        out_type=(
            jax.ShapeDtypeStruct((BATCH, DIM), jnp.float32),
            jax.ShapeDtypeStruct((BATCH,), jnp.float32),
        ),
        mesh=mesh,
        scratch_types=(
            [pltpu.VMEM((DIM,), jnp.int32),
             pltpu.VMEM((ROWS_PER_W,), jnp.float32)]
            + [pltpu.VMEM((CHUNK_R, DIM), jnp.float32)] * (2 * NBUF)
            + [pltpu.SemaphoreType.DMA] * (2 * NBUF)
        ),
        compiler_params=pltpu.CompilerParams(
            use_tc_tiling_on_sc=True, needs_layout_passes=False
        ),
    )
    return f(x, perm)


def kernel(x, perm):
    out, logdet = _permute(x, perm.astype(jnp.int32))
    return (out, logdet)
